# baseline (speedup 1.0000x reference)
import jax
import jax.numpy as jnp
from jax import lax
from jax.experimental import pallas as pl
from jax.experimental.pallas import tpu as pltpu
from jax.experimental.pallas import tpu_sc as plsc

N_NODES = 10000
N_EDGES = 320000
D = 128

NC = 2
NS = 16
NW = NC * NS
CHUNK = 32
NROWS = N_EDGES // CHUNK
NCHUNK = 320
NSTAGE = 8
NBUF = 4
SCHUNK = NCHUNK // NSTAGE
SQUAD = SCHUNK // NBUF
LAST_W = NW - 1
LAST_NSTAGE = (NROWS - LAST_W * NCHUNK) // SCHUNK

ZROWS = 16
SLAB = 624

_BLK = 2000


def _memset_zero(ref, nrows, ncols):
    z16 = jnp.zeros((16,), jnp.float32)
    per_row = ncols // 16

    def body(i, _):
        ref[i // per_row, pl.ds((i % per_row) * 16, 16)] = z16
        return 0

    lax.fori_loop(0, nrows * per_row, body, 0)


def _deg_kernel_body(dst_hbm, deg_out, deg_shared, dst_v, ones_v, zbuf_v, dsem):
    c = lax.axis_index("c")
    s = lax.axis_index("s")
    wid = c * NS + s
    base = wid * NCHUNK
    nst = jnp.where(wid == LAST_W, LAST_NSTAGE, NSTAGE)

    @pl.when(s == 0)
    def _init():
        z16 = jnp.zeros((16,), jnp.float32)

        def zb(i, _):
            zbuf_v[pl.ds(i * 16, 16)] = z16
            return 0

        lax.fori_loop(0, N_NODES // 16, zb, 0)
        pltpu.sync_copy(zbuf_v, deg_shared)

    o16 = jnp.ones((16,), jnp.float32)

    def fill(i, _):
        ones_v[pl.ds(i * 16, 16)] = o16
        return 0

    lax.fori_loop(0, CHUNK // 16, fill, 0)
    plsc.subcore_barrier()

    for g in range(NSTAGE):
        @pl.when(g < nst)
        def _run(g=g):
            pltpu.sync_copy(dst_hbm.at[pl.ds(base + g * SCHUNK, SCHUNK)], dst_v)

            def fire(j, _):
                pltpu.async_copy(ones_v, deg_shared.at[dst_v.at[j]], dsem,
                                 add=True)
                return 0

            lax.fori_loop(0, SCHUNK, fire, 0)

            def drain(j, _):
                pltpu.make_async_copy(ones_v, deg_shared.at[dst_v.at[0]],
                                      dsem).wait()
                return 0

            lax.fori_loop(0, SCHUNK, drain, 0)

    plsc.subcore_barrier()

    @pl.when(s == 0)
    def _readback():
        pltpu.sync_copy(deg_shared, deg_out.at[c])


def _deg_partials(dst_rows):
    mesh = plsc.VectorSubcoreMesh(core_axis_name="c", subcore_axis_name="s")
    return pl.kernel(
        _deg_kernel_body,
        out_type=jax.ShapeDtypeStruct((NC, N_NODES), jnp.float32),
        mesh=mesh,
        scratch_types=[
            pltpu.VMEM_SHARED((N_NODES,), jnp.float32),
            pltpu.VMEM((SCHUNK, CHUNK), jnp.int32),
            pltpu.VMEM((CHUNK,), jnp.float32),
            pltpu.VMEM((N_NODES,), jnp.float32),
            pltpu.SemaphoreType.DMA,
        ],
    )(dst_rows)


def _main_kernel_body(h2_hbm, src_hbm, dst_hbm, acc_out,
                      acc_shared, src_a, dst_a, src_b, dst_b,
                      rows_0, rows_1, rows_2, rows_3, zbuf_v,
                      gsem_0, gsem_1, gsem_2, gsem_3,
                      ssem_0, ssem_1, ssem_2, ssem_3, isem_a, isem_b):
    c = lax.axis_index("c")
    s = lax.axis_index("s")
    wid = c * NS + s
    base = wid * NCHUNK
    full = wid < LAST_W

    _memset_zero(zbuf_v, ZROWS, D)

    def zslab(i, _):
        pltpu.sync_copy(zbuf_v, acc_shared.at[pl.ds(s * SLAB + i * ZROWS, ZROWS)])
        return 0

    lax.fori_loop(0, SLAB // ZROWS, zslab, 0)

    @pl.when(s == 0)
    def _ztail():
        pltpu.sync_copy(zbuf_v, acc_shared.at[pl.ds(NS * SLAB, ZROWS)])

    idx_bufs = ((src_a, dst_a, isem_a), (src_b, dst_b, isem_b))

    def _stage(t, bufs):
        si, di, isem = bufs
        pltpu.async_copy(src_hbm.at[pl.ds(base + t * SCHUNK, SCHUNK)], si, isem)
        pltpu.async_copy(dst_hbm.at[pl.ds(base + t * SCHUNK, SCHUNK)], di, isem)

    def _wait_stage(bufs):
        si, di, isem = bufs
        pltpu.make_async_copy(src_hbm.at[pl.ds(0, SCHUNK)], si, isem).wait()
        pltpu.make_async_copy(dst_hbm.at[pl.ds(0, SCHUNK)], di, isem).wait()

    _stage(0, idx_bufs[0])
    plsc.subcore_barrier()

    rows = (rows_0, rows_1, rows_2, rows_3)
    gsems = (gsem_0, gsem_1, gsem_2, gsem_3)
    ssems = (ssem_0, ssem_1, ssem_2, ssem_3)

    def _run_stage(t):
        si, di, _ = idx_bufs[t % 2]
        _wait_stage(idx_bufs[t % 2])
        if t < NSTAGE - 1:
            if t + 1 < LAST_NSTAGE:
                _stage(t + 1, idx_bufs[(t + 1) % 2])
            else:
                @pl.when(full)
                def _pf():
                    _stage(t + 1, idx_bufs[(t + 1) % 2])

        for k in range(NBUF):
            pltpu.async_copy(h2_hbm.at[si.at[k]], rows[k], gsems[k])

        def quad(g, _, si=si, di=di):
            j0 = NBUF * g
            sdescs = []
            for k in range(NBUF):
                pltpu.make_async_copy(h2_hbm.at[si.at[j0 + k]], rows[k],
                                      gsems[k]).wait()
                sdescs.append(pltpu.async_copy(
                    rows[k], acc_shared.at[di.at[j0 + k]], ssems[k], add=True))
            for k in range(NBUF):
                sdescs[k].wait()

                @pl.when(g < SQUAD - 1)
                def _next(k=k):
                    pltpu.async_copy(h2_hbm.at[si.at[j0 + NBUF + k]],
                                     rows[k], gsems[k])

            return 0

        lax.fori_loop(0, SQUAD, quad, 0)

    for t in range(NSTAGE):
        if t < LAST_NSTAGE:
            _run_stage(t)
        else:
            @pl.when(full)
            def _rs(t=t):
                _run_stage(t)

    plsc.subcore_barrier()

    pltpu.sync_copy(acc_shared.at[pl.ds(s * SLAB, SLAB)],
                    acc_out.at[c, pl.ds(s * SLAB, SLAB)])

    @pl.when(s == 0)
    def _rtail():
        pltpu.sync_copy(acc_shared.at[pl.ds(NS * SLAB, ZROWS)],
                        acc_out.at[c, pl.ds(NS * SLAB, ZROWS)])


def _acc_partials(h2, src_rows, dst_rows):
    mesh = plsc.VectorSubcoreMesh(core_axis_name="c", subcore_axis_name="s")
    return pl.kernel(
        _main_kernel_body,
        out_type=jax.ShapeDtypeStruct((NC, N_NODES, D), jnp.float32),
        mesh=mesh,
        scratch_types=[
            pltpu.VMEM_SHARED((N_NODES, D), jnp.float32),
            pltpu.VMEM((SCHUNK, CHUNK), jnp.int32),
            pltpu.VMEM((SCHUNK, CHUNK), jnp.int32),
            pltpu.VMEM((SCHUNK, CHUNK), jnp.int32),
            pltpu.VMEM((SCHUNK, CHUNK), jnp.int32),
            pltpu.VMEM((CHUNK, D), jnp.float32),
            pltpu.VMEM((CHUNK, D), jnp.float32),
            pltpu.VMEM((CHUNK, D), jnp.float32),
            pltpu.VMEM((CHUNK, D), jnp.float32),
            pltpu.VMEM((ZROWS, D), jnp.float32),
            pltpu.SemaphoreType.DMA,
            pltpu.SemaphoreType.DMA,
            pltpu.SemaphoreType.DMA,
            pltpu.SemaphoreType.DMA,
            pltpu.SemaphoreType.DMA,
            pltpu.SemaphoreType.DMA,
            pltpu.SemaphoreType.DMA,
            pltpu.SemaphoreType.DMA,
            pltpu.SemaphoreType.DMA,
            pltpu.SemaphoreType.DMA,
        ],
    )(h2, src_rows, dst_rows)


def _h2_body(x_ref, w_ref, deg_ref, h2_ref):
    dis = lax.rsqrt(deg_ref[0] + deg_ref[1] + 1.0)
    h = jnp.dot(x_ref[...], w_ref[...], preferred_element_type=jnp.float32)
    h2_ref[...] = h * dis


def _h2_call(x, W_enc, deg3):
    grid = N_NODES // _BLK
    return pl.pallas_call(
        _h2_body,
        grid=(grid,),
        in_specs=[
            pl.BlockSpec((_BLK, D), lambda i: (i, 0)),
            pl.BlockSpec((D, D), lambda i: (0, 0)),
            pl.BlockSpec((NC, _BLK, 1), lambda i: (0, i, 0)),
        ],
        out_specs=pl.BlockSpec((_BLK, D), lambda i: (i, 0)),
        out_shape=jax.ShapeDtypeStruct((N_NODES, D), jnp.float32),
    )(x, W_enc, deg3)


def _dec_body(acc_ref, h2_ref, deg_ref, be_ref, wd_ref, bd_ref, out_ref):
    dis = lax.rsqrt(deg_ref[0] + deg_ref[1] + 1.0)
    a = acc_ref[0] + acc_ref[1] + h2_ref[...]
    emb = jnp.maximum(a * dis + be_ref[0, :], 0.0)
    out_ref[...] = (jnp.dot(emb, wd_ref[...], preferred_element_type=jnp.float32)
                    + bd_ref[0, :])


def _dec_call(acc2, h2, deg3, b_enc, W_dec, b_dec):
    grid = N_NODES // _BLK
    return pl.pallas_call(
        _dec_body,
        grid=(grid,),
        in_specs=[
            pl.BlockSpec((NC, _BLK, D), lambda i: (0, i, 0)),
            pl.BlockSpec((_BLK, D), lambda i: (i, 0)),
            pl.BlockSpec((NC, _BLK, 1), lambda i: (0, i, 0)),
            pl.BlockSpec((1, D), lambda i: (0, 0)),
            pl.BlockSpec((D, D), lambda i: (0, 0)),
            pl.BlockSpec((1, D), lambda i: (0, 0)),
        ],
        out_specs=pl.BlockSpec((_BLK, D), lambda i: (i, 0)),
        out_shape=jax.ShapeDtypeStruct((N_NODES, D), jnp.float32),
    )(acc2, h2, deg3, b_enc.reshape(1, D), W_dec, b_dec.reshape(1, D))


def kernel(x, edge_index, W_enc, b_enc, W_dec, b_dec):
    src = edge_index[0].astype(jnp.int32).reshape(NROWS, CHUNK)
    dst = edge_index[1].astype(jnp.int32).reshape(NROWS, CHUNK)

    deg2 = _deg_partials(dst)
    deg3 = deg2.reshape(NC, N_NODES, 1)
    h2 = _h2_call(x, W_enc, deg3)
    acc2 = _acc_partials(h2, src, dst)
    return _dec_call(acc2, h2, deg3, b_enc, W_dec, b_dec)


# speedup vs baseline: 1.0190x; 1.0190x over previous
import jax
import jax.numpy as jnp
from jax import lax
from jax.experimental import pallas as pl
from jax.experimental.pallas import tpu as pltpu
from jax.experimental.pallas import tpu_sc as plsc

N_NODES = 10000
N_PAD = 10016
N_EDGES = 320000
D = 128

NC = 2
NS = 16
NW = NC * NS
CHUNK = 32
NCHUNK = 320
NSTAGE = 8
NBUF = 4
SCHUNK = NCHUNK // NSTAGE
SQUAD = SCHUNK // NBUF
E_PAD = NW * NCHUNK * CHUNK
ZROWS = 16
SLAB = 624

import numpy as _np
_NPAD_E = E_PAD - N_EDGES
_PAD_SRC = _np.asarray((_np.arange(_NPAD_E) * 13) % N_NODES, _np.int32)
_PAD_DST = _np.asarray(N_NODES + _np.arange(_NPAD_E) % (N_PAD - N_NODES),
                       _np.int32)


def _memset_zero(ref, nrows, ncols):
    z16 = jnp.zeros((16,), jnp.float32)
    per_row = ncols // 16

    def body(i, _):
        ref[i // per_row, pl.ds((i % per_row) * 16, 16)] = z16
        return 0

    lax.fori_loop(0, nrows * per_row, body, 0)


def _deg_kernel_body(dst_hbm, deg_out, deg_shared, dst_v, ones_v, zbuf_v, dsem):
    c = lax.axis_index("c")
    s = lax.axis_index("s")
    wid = c * NS + s

    @pl.when(s == 0)
    def _init():
        z16 = jnp.zeros((16,), jnp.float32)

        def zb(i, _):
            zbuf_v[pl.ds(i * 16, 16)] = z16
            return 0

        lax.fori_loop(0, N_PAD // 16, zb, 0)
        pltpu.sync_copy(zbuf_v, deg_shared)

    o16 = jnp.ones((16,), jnp.float32)

    def fill(i, _):
        ones_v[pl.ds(i * 16, 16)] = o16
        return 0

    lax.fori_loop(0, CHUNK // 16, fill, 0)

    pltpu.sync_copy(dst_hbm.at[wid], dst_v)
    plsc.subcore_barrier()

    def chunk(j, _):
        pltpu.async_copy(ones_v, deg_shared.at[dst_v.at[j]], dsem, add=True)
        return 0

    lax.fori_loop(0, NCHUNK, chunk, 0)

    def drain(j, _):
        pltpu.make_async_copy(ones_v, deg_shared.at[dst_v.at[0]], dsem).wait()
        return 0

    lax.fori_loop(0, NCHUNK, drain, 0)
    plsc.subcore_barrier()

    @pl.when(s == 0)
    def _readback():
        pltpu.sync_copy(deg_shared, deg_out.at[c])


def _deg_partials(dst_grouped):
    mesh = plsc.VectorSubcoreMesh(core_axis_name="c", subcore_axis_name="s")
    return pl.kernel(
        _deg_kernel_body,
        out_type=jax.ShapeDtypeStruct((NC, N_PAD), jnp.float32),
        mesh=mesh,
        scratch_types=[
            pltpu.VMEM_SHARED((N_PAD,), jnp.float32),
            pltpu.VMEM((NCHUNK, CHUNK), jnp.int32),
            pltpu.VMEM((CHUNK,), jnp.float32),
            pltpu.VMEM((N_PAD,), jnp.float32),
            pltpu.SemaphoreType.DMA,
        ],
    )(dst_grouped)


def _main_kernel_body(h2_hbm, src_hbm, dst_hbm, acc_out,
                      acc_shared, src_a, dst_a, src_b, dst_b,
                      rows_0, rows_1, rows_2, rows_3, zbuf_v,
                      gsem_0, gsem_1, gsem_2, gsem_3,
                      ssem_0, ssem_1, ssem_2, ssem_3, isem_a, isem_b):
    c = lax.axis_index("c")
    s = lax.axis_index("s")
    wid = c * NS + s

    _memset_zero(zbuf_v, ZROWS, D)

    def zslab(i, _):
        pltpu.sync_copy(zbuf_v, acc_shared.at[pl.ds(s * SLAB + i * ZROWS, ZROWS)])
        return 0

    lax.fori_loop(0, SLAB // ZROWS, zslab, 0)

    @pl.when(s == 0)
    def _ztail():
        def zt(i, _):
            pltpu.sync_copy(zbuf_v,
                            acc_shared.at[pl.ds(NS * SLAB + i * ZROWS, ZROWS)])
            return 0

        lax.fori_loop(0, (N_PAD - NS * SLAB) // ZROWS, zt, 0)

    idx_bufs = ((src_a, dst_a, isem_a), (src_b, dst_b, isem_b))

    def _stage(t, bufs):
        si, di, isem = bufs
        pltpu.async_copy(src_hbm.at[wid, pl.ds(t * SCHUNK, SCHUNK)], si, isem)
        pltpu.async_copy(dst_hbm.at[wid, pl.ds(t * SCHUNK, SCHUNK)], di, isem)

    def _wait_stage(bufs):
        si, di, isem = bufs
        pltpu.make_async_copy(src_hbm.at[wid, pl.ds(0, SCHUNK)], si, isem).wait()
        pltpu.make_async_copy(dst_hbm.at[wid, pl.ds(0, SCHUNK)], di, isem).wait()

    _stage(0, idx_bufs[0])
    plsc.subcore_barrier()

    rows = (rows_0, rows_1, rows_2, rows_3)
    gsems = (gsem_0, gsem_1, gsem_2, gsem_3)
    ssems = (ssem_0, ssem_1, ssem_2, ssem_3)

    for t in range(NSTAGE):
        si, di, _ = idx_bufs[t % 2]
        _wait_stage(idx_bufs[t % 2])
        if t < NSTAGE - 1:
            _stage(t + 1, idx_bufs[(t + 1) % 2])

        for k in range(NBUF):
            pltpu.async_copy(h2_hbm.at[si.at[k]], rows[k], gsems[k])

        def quad(g, _, si=si, di=di):
            j0 = NBUF * g
            sdescs = []
            for k in range(NBUF):
                pltpu.make_async_copy(h2_hbm.at[si.at[j0 + k]], rows[k],
                                      gsems[k]).wait()
                sdescs.append(pltpu.async_copy(
                    rows[k], acc_shared.at[di.at[j0 + k]], ssems[k], add=True))
            for k in range(NBUF):
                sdescs[k].wait()

                @pl.when(g < SQUAD - 1)
                def _next(k=k):
                    pltpu.async_copy(h2_hbm.at[si.at[j0 + NBUF + k]],
                                     rows[k], gsems[k])

            return 0

        lax.fori_loop(0, SQUAD, quad, 0)

    plsc.subcore_barrier()

    pltpu.sync_copy(acc_shared.at[pl.ds(s * SLAB, SLAB)],
                    acc_out.at[c, pl.ds(s * SLAB, SLAB)])

    @pl.when(s == 0)
    def _rtail():
        pltpu.sync_copy(acc_shared.at[pl.ds(NS * SLAB, N_PAD - NS * SLAB)],
                        acc_out.at[c, pl.ds(NS * SLAB, N_PAD - NS * SLAB)])


def _acc_partials(h2, src_grouped, dst_grouped):
    mesh = plsc.VectorSubcoreMesh(core_axis_name="c", subcore_axis_name="s")
    return pl.kernel(
        _main_kernel_body,
        out_type=jax.ShapeDtypeStruct((NC, N_PAD, D), jnp.float32),
        mesh=mesh,
        scratch_types=[
            pltpu.VMEM_SHARED((N_PAD, D), jnp.float32),
            pltpu.VMEM((SCHUNK, CHUNK), jnp.int32),
            pltpu.VMEM((SCHUNK, CHUNK), jnp.int32),
            pltpu.VMEM((SCHUNK, CHUNK), jnp.int32),
            pltpu.VMEM((SCHUNK, CHUNK), jnp.int32),
            pltpu.VMEM((CHUNK, D), jnp.float32),
            pltpu.VMEM((CHUNK, D), jnp.float32),
            pltpu.VMEM((CHUNK, D), jnp.float32),
            pltpu.VMEM((CHUNK, D), jnp.float32),
            pltpu.VMEM((ZROWS, D), jnp.float32),
            pltpu.SemaphoreType.DMA,
            pltpu.SemaphoreType.DMA,
            pltpu.SemaphoreType.DMA,
            pltpu.SemaphoreType.DMA,
            pltpu.SemaphoreType.DMA,
            pltpu.SemaphoreType.DMA,
            pltpu.SemaphoreType.DMA,
            pltpu.SemaphoreType.DMA,
            pltpu.SemaphoreType.DMA,
            pltpu.SemaphoreType.DMA,
        ],
    )(h2, src_grouped, dst_grouped)


_BLK = 2000


def _h2_body(x_ref, w_ref, deg_ref, h2_ref):
    dis = lax.rsqrt(deg_ref[0] + deg_ref[1] + 1.0)
    h = jnp.dot(x_ref[...], w_ref[...], preferred_element_type=jnp.float32)
    h2_ref[...] = h * dis


def _h2_call(x, W_enc, deg3):
    grid = N_NODES // _BLK
    return pl.pallas_call(
        _h2_body,
        grid=(grid,),
        in_specs=[
            pl.BlockSpec((_BLK, D), lambda i: (i, 0)),
            pl.BlockSpec((D, D), lambda i: (0, 0)),
            pl.BlockSpec((NC, _BLK, 1), lambda i: (0, i, 0)),
        ],
        out_specs=pl.BlockSpec((_BLK, D), lambda i: (i, 0)),
        out_shape=jax.ShapeDtypeStruct((N_NODES, D), jnp.float32),
    )(x, W_enc, deg3)


def _dec_body(acc_ref, h2_ref, deg_ref, be_ref, wd_ref, bd_ref, out_ref):
    dis = lax.rsqrt(deg_ref[0] + deg_ref[1] + 1.0)
    a = acc_ref[0] + acc_ref[1] + h2_ref[...]
    emb = jnp.maximum(a * dis + be_ref[0, :], 0.0)
    out_ref[...] = (jnp.dot(emb, wd_ref[...], preferred_element_type=jnp.float32)
                    + bd_ref[0, :])


def _dec_call(acc2, h2, deg3, b_enc, W_dec, b_dec):
    grid = N_NODES // _BLK
    return pl.pallas_call(
        _dec_body,
        grid=(grid,),
        in_specs=[
            pl.BlockSpec((NC, _BLK, D), lambda i: (0, i, 0)),
            pl.BlockSpec((_BLK, D), lambda i: (i, 0)),
            pl.BlockSpec((NC, _BLK, 1), lambda i: (0, i, 0)),
            pl.BlockSpec((1, D), lambda i: (0, 0)),
            pl.BlockSpec((D, D), lambda i: (0, 0)),
            pl.BlockSpec((1, D), lambda i: (0, 0)),
        ],
        out_specs=pl.BlockSpec((_BLK, D), lambda i: (i, 0)),
        out_shape=jax.ShapeDtypeStruct((N_NODES, D), jnp.float32),
    )(acc2, h2, deg3, b_enc.reshape(1, D), W_dec, b_dec.reshape(1, D))


def kernel(x, edge_index, W_enc, b_enc, W_dec, b_dec):
    src = edge_index[0].astype(jnp.int32)
    dst = edge_index[1].astype(jnp.int32)
    src = jnp.concatenate([src, _PAD_SRC]).reshape(NW, NCHUNK, CHUNK)
    dst = jnp.concatenate([dst, _PAD_DST]).reshape(NW, NCHUNK, CHUNK)

    deg2 = _deg_partials(dst)
    deg3 = deg2.reshape(NC, N_PAD, 1)
    h2 = _h2_call(x, W_enc, deg3)
    acc2 = _acc_partials(h2, src, dst)
    return _dec_call(acc2, h2, deg3, b_enc, W_dec, b_dec)
